# baseline (speedup 1.0000x reference)
import jax
import jax.numpy as jnp
from jax import lax
from jax.experimental import pallas as pl
from jax.experimental.pallas import tpu as pltpu
from jax.experimental.pallas import tpu_sc as plsc
from jax.scipy.linalg import block_diag

N = 100000
E = 1600000
F = 39
H = 32
EP = E // 4
NCHUNK = 5
CEP = EP // NCHUNK

NC = 2
NS = 16
NW = NC * NS

BLK = 512
PR = BLK // 4
NBLK = CEP // PR
SCB = 8
NSC = NBLK // SCB
TAIL = NBLK - NSC * SCB
G_ITERS = -(-NSC // NW)
HALF = N // NC
PAD = 64
AROWS = HALF + PAD

BLKS = 256
PRS = BLKS // 4
SCBS = 8
NBLKS = CEP // PRS
NSCS = NBLKS // SCBS
TAILS = NBLKS - NSCS * SCBS
S_ITERS = -(-NSCS // NS)

_mesh = plsc.VectorSubcoreMesh(
    core_axis_name="c", subcore_axis_name="s", num_cores=NC, num_subcores=NS)
_sc_params = pltpu.CompilerParams(use_tc_tiling_on_sc=False)


def _node_proj_body(n_ref, wsrc_ref, wdst_ref, a_ref, b_ref):
    x = n_ref[...]
    a_ref[...] = jnp.dot(x, wsrc_ref[...], preferred_element_type=jnp.float32)
    b_ref[...] = jnp.dot(x, wdst_ref[...], preferred_element_type=jnp.float32)


def _edge_proj_body(e0_ref, e1_ref, e2_ref, e3_ref, w_ref, b_ref, c_ref):
    for j, e_ref in enumerate((e0_ref, e1_ref, e2_ref, e3_ref)):
        c_ref[:, j * H:(j + 1) * H] = (
            jnp.dot(e_ref[...], w_ref[...], preferred_element_type=jnp.float32)
            + b_ref[...])


def _edge_mlp2_body(pre_ref, w_ref, b_ref, o_ref):
    h = jnp.tanh(pre_ref[...])
    o_ref[...] = jnp.tanh(
        jnp.dot(h, w_ref[...], preferred_element_type=jnp.float32) + b_ref[...])


def _node_mlp_body(nu0_ref, nu1_ref, nu2_ref, nu3_ref, nu4_ref, n_ref,
                   w1a_ref, w1b_ref, b1_ref, w2_ref, b2_ref, o_ref):
    nu = (nu0_ref[...] + nu1_ref[...] + nu2_ref[...] + nu3_ref[...]
          + nu4_ref[...])
    z = jnp.tanh(
        jnp.dot(nu, w1a_ref[...], preferred_element_type=jnp.float32)
        + jnp.dot(n_ref[...], w1b_ref[...], preferred_element_type=jnp.float32)
        + b1_ref[...])
    o_ref[...] = (
        jnp.dot(z, w2_ref[...], preferred_element_type=jnp.float32)
        + b2_ref[...])


def _make_gather_add_body(r0):
    def _gather_add_body(a_hbm, b_hbm, c_hbm, src_hbm, dst_hbm, pre_hbm,
                         ixs, ixd, a0, a1, b0, b1, c0, c1, c2,
                         isem, g0sem, g1sem, s0sem, s1sem, s2sem):
        wid = lax.axis_index("s") * NC + lax.axis_index("c")
        a_bufs = (a0, a1)
        b_bufs = (b0, b1)
        c_bufs = (c0, c1, c2)
        gsems = (g0sem, g1sem)
        ssems = (s0sem, s1sem, s2sem)

        def vadd(c_v, a_v, b_v):
            def add_body(i, carry2):
                for h in range(2):
                    sl2 = pl.ds(h * 16, 16)
                    c_v[i, sl2] = c_v[i, sl2] + a_v[i, sl2] + b_v[i, sl2]
                return carry2
            lax.fori_loop(0, BLK, add_body, 0, unroll=4)

        def sc_body(t, carry):
            scid = wid + NW * t

            @pl.when(scid < NSC)
            def _():
                row0 = scid * (SCB * PR)
                ids = []
                for j in range(4):
                    ids.append(pltpu.async_copy(
                        src_hbm.at[pl.ds(j * EP + r0 + row0, SCB * PR)],
                        ixs.at[j], isem))
                    ids.append(pltpu.async_copy(
                        dst_hbm.at[pl.ds(j * EP + r0 + row0, SCB * PR)],
                        ixd.at[j], isem))
                for d in ids:
                    d.wait()

                pend_store = [None, None, None]

                def fire_blk(b):
                    q2, q3 = b % 2, b % 3
                    if pend_store[q3] is not None:
                        for d in pend_store[q3]:
                            d.wait()
                        pend_store[q3] = None
                    rb = row0 + b * PR
                    ds = []
                    for j in range(4):
                        sl = pl.ds(j * PR, PR)
                        ds.append(pltpu.async_copy(
                            a_hbm.at[ixs.at[j, pl.ds(b * PR, PR)]],
                            a_bufs[q2].at[sl], gsems[q2]))
                        ds.append(pltpu.async_copy(
                            b_hbm.at[ixd.at[j, pl.ds(b * PR, PR)]],
                            b_bufs[q2].at[sl], gsems[q2]))
                        ds.append(pltpu.async_copy(
                            c_hbm.at[pl.ds(rb, PR), pl.ds(j * H, H)],
                            c_bufs[q3].at[sl], gsems[q2]))
                    return ds

                pend_g = [fire_blk(0), fire_blk(1)]
                for b in range(SCB):
                    q2, q3 = b % 2, b % 3
                    for d in pend_g[q2]:
                        d.wait()
                    vadd(c_bufs[q3], a_bufs[q2], b_bufs[q2])
                    rb = row0 + b * PR
                    pend_store[q3] = [pltpu.async_copy(
                        c_bufs[q3].at[pl.ds(j * PR, PR)],
                        pre_hbm.at[pl.ds(rb, PR), pl.ds(j * H, H)],
                        ssems[q3]) for j in range(4)]
                    if b + 2 < SCB:
                        pend_g[q2] = fire_blk(b + 2)
                for q3 in range(3):
                    if pend_store[q3] is not None:
                        for d in pend_store[q3]:
                            d.wait()

            return carry

        lax.fori_loop(0, G_ITERS, sc_body, 0)

        @pl.when(wid < TAIL)
        def _():
            g = NSC * SCB + wid
            rb = g * PR
            for j in range(4):
                pltpu.sync_copy(src_hbm.at[pl.ds(j * EP + r0 + rb, PR)],
                                ixs.at[j, pl.ds(0, PR)])
                pltpu.sync_copy(dst_hbm.at[pl.ds(j * EP + r0 + rb, PR)],
                                ixd.at[j, pl.ds(0, PR)])
            ds = []
            for j in range(4):
                sl = pl.ds(j * PR, PR)
                ds.append(pltpu.async_copy(
                    a_hbm.at[ixs.at[j, pl.ds(0, PR)]], a0.at[sl], g0sem))
                ds.append(pltpu.async_copy(
                    b_hbm.at[ixd.at[j, pl.ds(0, PR)]], b0.at[sl], g0sem))
                ds.append(pltpu.async_copy(
                    c_hbm.at[pl.ds(rb, PR), pl.ds(j * H, H)], c0.at[sl],
                    g0sem))
            for d in ds:
                d.wait()

            def add_body(i, carry2):
                for h in range(2):
                    sl2 = pl.ds(h * 16, 16)
                    c0[i, sl2] = c0[i, sl2] + a0[i, sl2] + b0[i, sl2]
                return carry2
            lax.fori_loop(0, BLK, add_body, 0, unroll=4)
            for j in range(4):
                pltpu.sync_copy(c0.at[pl.ds(j * PR, PR)],
                                pre_hbm.at[pl.ds(rb, PR), pl.ds(j * H, H)])

    return _gather_add_body


def _make_scatter_add_body(r0):
    def _scatter_add_body(eup_hbm, src_hbm, nup_hbm,
                          ixs, l0, l1, l2, e0, e1, e2, accum,
                          isem, d0sem, d1sem, d2sem, c0sem, c1sem, c2sem):
        cid = lax.axis_index("c")
        sid = lax.axis_index("s")
        base_node = cid * HALF
        e_vs = (e0, e1, e2)
        lidx = (l0, l1, l2)
        lsems = (d0sem, d1sem, d2sem)
        scsems = (c0sem, c1sem, c2sem)

        def z_body(i, carry):
            zero = jnp.zeros((16,), jnp.float32)
            e0[i, pl.ds(0, 16)] = zero
            e0[i, pl.ds(16, 16)] = zero
            return carry

        lax.fori_loop(0, BLKS, z_body, 0, unroll=8)
        rows_per_s = AROWS // NS
        zbase = sid * rows_per_s
        done = 0
        while done < rows_per_s:
            chunk = min(BLKS, rows_per_s - done)
            pltpu.sync_copy(e0.at[pl.ds(0, chunk)],
                            accum.at[pl.ds(zbase + done, chunk)])
            done += chunk
        plsc.subcore_barrier()

        def transform(ib, q):
            def tr_body(k, carry):
                for u in range(2):
                    kk = k * 2 + u
                    j, r = kk // 4, kk % 4
                    v = ixs[j, pl.ds(ib * PRS + r * 16, 16)]
                    li = v - base_node
                    oob = (li < 0) | (li >= HALF)
                    dummy = HALF + (v & (PAD - 1))
                    lidx[q][j, pl.ds(r * 16, 16)] = jnp.where(oob, dummy, li)
                return carry
            lax.fori_loop(0, 8, tr_body, 0)

        def sc_body(t, carry):
            scid = sid + NS * t

            @pl.when(scid < NSCS)
            def _():
                row0 = scid * (SCBS * PRS)
                ids = [pltpu.async_copy(
                    src_hbm.at[pl.ds(j * EP + r0 + row0, SCBS * PRS)],
                    ixs.at[j], isem)
                    for j in range(4)]
                for d in ids:
                    d.wait()

                pend_sc = [None, None, None]

                def fire_load(b):
                    q = b % 3
                    if pend_sc[q] is not None:
                        for d in pend_sc[q]:
                            d.wait()
                        pend_sc[q] = None
                    rb = row0 + b * PRS
                    return [pltpu.async_copy(
                        eup_hbm.at[pl.ds(rb, PRS), pl.ds(j * H, H)],
                        e_vs[q].at[pl.ds(j * PRS, PRS)], lsems[q])
                        for j in range(4)]

                pend_l = [fire_load(0), fire_load(1), None]
                for b in range(SCBS):
                    q = b % 3
                    for d in pend_l[q]:
                        d.wait()
                    transform(b, q)
                    pend_sc[q] = [pltpu.async_copy(
                        e_vs[q].at[pl.ds(j * PRS, PRS)],
                        accum.at[lidx[q].at[j]], scsems[q], add=True)
                        for j in range(4)]
                    if b + 2 < SCBS:
                        pend_l[(b + 2) % 3] = fire_load(b + 2)
                for q in range(3):
                    if pend_sc[q] is not None:
                        for d in pend_sc[q]:
                            d.wait()

            return carry

        lax.fori_loop(0, S_ITERS, sc_body, 0)

        @pl.when(sid < TAILS)
        def _():
            g = NSCS * SCBS + sid
            rb = g * PRS
            for j in range(4):
                pltpu.sync_copy(src_hbm.at[pl.ds(j * EP + r0 + rb, PRS)],
                                ixs.at[j, pl.ds(0, PRS)])
                pltpu.sync_copy(eup_hbm.at[pl.ds(rb, PRS), pl.ds(j * H, H)],
                                e0.at[pl.ds(j * PRS, PRS)])
            transform(0, 0)
            for j in range(4):
                pltpu.sync_copy(e0.at[pl.ds(j * PRS, PRS)],
                                accum.at[l0.at[j]], add=True)

        plsc.subcore_barrier()

        rows_out = HALF // NS
        obase = sid * rows_out
        pltpu.sync_copy(accum.at[pl.ds(obase, rows_out)],
                        nup_hbm.at[pl.ds(base_node + obase, rows_out)])

    return _scatter_add_body


def _tc_call(body, grid, in_specs, out_specs, out_shape):
    return pl.pallas_call(
        body, grid=grid, in_specs=in_specs, out_specs=out_specs,
        out_shape=out_shape)


def kernel(n, e, e_i, batch, We1, be1, We2, be2, Wn1, bn1, Wn2, bn2):
    del batch
    src = e_i[0]
    dst = e_i[1]
    W2bd = block_diag(*([We2] * 4))
    b2t = jnp.tile(be2, 4).reshape(1, 128)

    BN = 2000
    A, B = _tc_call(
        _node_proj_body, (N // BN,),
        [pl.BlockSpec((BN, F), lambda i: (i, 0)),
         pl.BlockSpec((F, H), lambda i: (0, 0)),
         pl.BlockSpec((F, H), lambda i: (0, 0))],
        [pl.BlockSpec((BN, H), lambda i: (i, 0)),
         pl.BlockSpec((BN, H), lambda i: (i, 0))],
        [jax.ShapeDtypeStruct((N, H), jnp.float32),
         jax.ShapeDtypeStruct((N, H), jnp.float32)])(
            n, We1[0:F], We1[F + 10:])

    BE4 = 4000
    NB4 = EP // BE4
    CB4 = CEP // BE4
    be1r = be1.reshape(1, H)
    We1m = We1[F:F + 10]

    gather_scratch = [
        pltpu.VMEM((4, SCB * PR), jnp.int32),
        pltpu.VMEM((4, SCB * PR), jnp.int32),
        pltpu.VMEM((BLK, H), jnp.float32),
        pltpu.VMEM((BLK, H), jnp.float32),
        pltpu.VMEM((BLK, H), jnp.float32),
        pltpu.VMEM((BLK, H), jnp.float32),
        pltpu.VMEM((BLK, H), jnp.float32),
        pltpu.VMEM((BLK, H), jnp.float32),
        pltpu.VMEM((BLK, H), jnp.float32),
        pltpu.SemaphoreType.DMA,
        pltpu.SemaphoreType.DMA,
        pltpu.SemaphoreType.DMA,
        pltpu.SemaphoreType.DMA,
        pltpu.SemaphoreType.DMA,
        pltpu.SemaphoreType.DMA,
    ]
    scatter_scratch = [
        pltpu.VMEM((4, SCBS * PRS), jnp.int32),
        pltpu.VMEM((4, PRS), jnp.int32),
        pltpu.VMEM((4, PRS), jnp.int32),
        pltpu.VMEM((4, PRS), jnp.int32),
        pltpu.VMEM((BLKS, H), jnp.float32),
        pltpu.VMEM((BLKS, H), jnp.float32),
        pltpu.VMEM((BLKS, H), jnp.float32),
        pltpu.VMEM_SHARED((AROWS, H), jnp.float32),
        pltpu.SemaphoreType.DMA,
        pltpu.SemaphoreType.DMA,
        pltpu.SemaphoreType.DMA,
        pltpu.SemaphoreType.DMA,
        pltpu.SemaphoreType.DMA,
        pltpu.SemaphoreType.DMA,
        pltpu.SemaphoreType.DMA,
    ]

    nups = []
    for k in range(NCHUNK):
        C4 = _tc_call(
            _edge_proj_body, (CB4,),
            [pl.BlockSpec((BE4, 10), lambda i, k=k: (k * CB4 + i, 0)),
             pl.BlockSpec((BE4, 10), lambda i, k=k: (NB4 + k * CB4 + i, 0)),
             pl.BlockSpec((BE4, 10),
                          lambda i, k=k: (2 * NB4 + k * CB4 + i, 0)),
             pl.BlockSpec((BE4, 10),
                          lambda i, k=k: (3 * NB4 + k * CB4 + i, 0)),
             pl.BlockSpec((10, H), lambda i: (0, 0)),
             pl.BlockSpec((1, H), lambda i: (0, 0))],
            pl.BlockSpec((BE4, 128), lambda i: (i, 0)),
            jax.ShapeDtypeStruct((CEP, 128), jnp.float32))(
                e, e, e, e, We1m, be1r)

        gather_add = pl.kernel(
            _make_gather_add_body(k * CEP),
            out_type=jax.ShapeDtypeStruct((CEP, 128), jnp.float32),
            mesh=_mesh,
            compiler_params=_sc_params,
            scratch_types=gather_scratch)
        pre4 = gather_add(A, B, C4, src, dst)

        e_up4 = _tc_call(
            _edge_mlp2_body, (CB4,),
            [pl.BlockSpec((BE4, 128), lambda i: (i, 0)),
             pl.BlockSpec((128, 128), lambda i: (0, 0)),
             pl.BlockSpec((1, 128), lambda i: (0, 0))],
            pl.BlockSpec((BE4, 128), lambda i: (i, 0)),
            jax.ShapeDtypeStruct((CEP, 128), jnp.float32))(
                pre4, W2bd, b2t)

        scatter = pl.kernel(
            _make_scatter_add_body(k * CEP),
            out_type=jax.ShapeDtypeStruct((N, H), jnp.float32),
            mesh=_mesh,
            compiler_params=_sc_params,
            scratch_types=scatter_scratch)
        nups.append(scatter(e_up4, src))

    out = _tc_call(
        _node_mlp_body, (N // BN,),
        [pl.BlockSpec((BN, H), lambda i: (i, 0)),
         pl.BlockSpec((BN, H), lambda i: (i, 0)),
         pl.BlockSpec((BN, H), lambda i: (i, 0)),
         pl.BlockSpec((BN, H), lambda i: (i, 0)),
         pl.BlockSpec((BN, H), lambda i: (i, 0)),
         pl.BlockSpec((BN, F), lambda i: (i, 0)),
         pl.BlockSpec((H, H), lambda i: (0, 0)),
         pl.BlockSpec((F, H), lambda i: (0, 0)),
         pl.BlockSpec((1, H), lambda i: (0, 0)),
         pl.BlockSpec((H, 1), lambda i: (0, 0)),
         pl.BlockSpec((1, 1), lambda i: (0, 0))],
        pl.BlockSpec((BN, 1), lambda i: (i, 0)),
        jax.ShapeDtypeStruct((N, 1), jnp.float32))(
            nups[0], nups[1], nups[2], nups[3], nups[4], n,
            Wn1[0:H], Wn1[H:], bn1.reshape(1, H), Wn2,
            bn2.reshape(1, 1))
    return out


# speedup vs baseline: 1.3144x; 1.3144x over previous
import jax
import jax.numpy as jnp
from jax import lax
from jax.experimental import pallas as pl
from jax.experimental.pallas import tpu as pltpu
from jax.experimental.pallas import tpu_sc as plsc
from jax.scipy.linalg import block_diag

N = 100000
E = 1600000
F = 39
H = 32
EP = E // 4
NCHUNK = 5
CEP = EP // NCHUNK

NC = 2
NS = 16
NW = NC * NS

BLK = 512
PR = BLK // 4
NBLK = CEP // PR
SCB = 5
NSC = NBLK // SCB
G_ITERS = -(-NSC // NW)
HALF = N // NC
PAD = 64
AROWS = HALF + PAD

BLKS = 256
PRS = BLKS // 4
SCBS = 8
NBLKS = CEP // PRS
NSCS = NBLKS // SCBS
TAILS = NBLKS - NSCS * SCBS
S_ITERS = -(-NSCS // NS)

_mesh = plsc.VectorSubcoreMesh(
    core_axis_name="c", subcore_axis_name="s", num_cores=NC, num_subcores=NS)
_sc_params = pltpu.CompilerParams(use_tc_tiling_on_sc=False)


def _node_proj_body(n_ref, wsrc_ref, wdst_ref, a_ref, b_ref):
    x = n_ref[...]
    a_ref[...] = jnp.dot(x, wsrc_ref[...], preferred_element_type=jnp.float32)
    b_ref[...] = jnp.dot(x, wdst_ref[...], preferred_element_type=jnp.float32)


def _edge_proj_body(e0_ref, e1_ref, e2_ref, e3_ref, w_ref, b_ref, c_ref):
    for j, e_ref in enumerate((e0_ref, e1_ref, e2_ref, e3_ref)):
        c_ref[:, j * H:(j + 1) * H] = (
            jnp.dot(e_ref[...], w_ref[...], preferred_element_type=jnp.float32)
            + b_ref[...])


def _edge_mlp2_body(pre_ref, c_ref, w_ref, b_ref, o_ref):
    h = jnp.tanh(pre_ref[...] + c_ref[...])
    o_ref[...] = jnp.tanh(
        jnp.dot(h, w_ref[...], preferred_element_type=jnp.float32) + b_ref[...])


def _node_mlp_body(nu0_ref, nu1_ref, nu2_ref, nu3_ref, nu4_ref, n_ref,
                   w1a_ref, w1b_ref, b1_ref, w2_ref, b2_ref, o_ref):
    nu = (nu0_ref[...] + nu1_ref[...] + nu2_ref[...] + nu3_ref[...]
          + nu4_ref[...])
    z = jnp.tanh(
        jnp.dot(nu, w1a_ref[...], preferred_element_type=jnp.float32)
        + jnp.dot(n_ref[...], w1b_ref[...], preferred_element_type=jnp.float32)
        + b1_ref[...])
    o_ref[...] = (
        jnp.dot(z, w2_ref[...], preferred_element_type=jnp.float32)
        + b2_ref[...])


def _make_gather_add_body(r0):
    def _gather_add_body(a_hbm, b_hbm, src_hbm, dst_hbm, pre_hbm,
                         ixs, ixd, a0, a1, b0, b1, c0, c1, c2,
                         isem, g0sem, g1sem, s0sem, s1sem, s2sem):
        wid = lax.axis_index("s") * NC + lax.axis_index("c")
        a_bufs = (a0, a1)
        b_bufs = (b0, b1)
        c_bufs = (c0, c1, c2)
        gsems = (g0sem, g1sem)
        ssems = (s0sem, s1sem, s2sem)

        def vadd(c_v, a_v, b_v):
            def add_body(i, carry2):
                for h in range(2):
                    sl2 = pl.ds(h * 16, 16)
                    c_v[i, sl2] = a_v[i, sl2] + b_v[i, sl2]
                return carry2
            lax.fori_loop(0, BLK, add_body, 0, unroll=4)

        def sc_body(t, carry):
            scid = wid + NW * t

            @pl.when(scid < NSC)
            def _():
                row0 = scid * (SCB * PR)
                ids = []
                for j in range(4):
                    ids.append(pltpu.async_copy(
                        src_hbm.at[pl.ds(j * EP + r0 + row0, SCB * PR)],
                        ixs.at[j], isem))
                    ids.append(pltpu.async_copy(
                        dst_hbm.at[pl.ds(j * EP + r0 + row0, SCB * PR)],
                        ixd.at[j], isem))
                for d in ids:
                    d.wait()

                pend_store = [None, None, None]

                def fire_blk(b):
                    q2 = b % 2
                    rb = row0 + b * PR
                    ds = []
                    for j in range(4):
                        sl = pl.ds(j * PR, PR)
                        ds.append(pltpu.async_copy(
                            a_hbm.at[ixs.at[j, pl.ds(b * PR, PR)]],
                            a_bufs[q2].at[sl], gsems[q2]))
                        ds.append(pltpu.async_copy(
                            b_hbm.at[ixd.at[j, pl.ds(b * PR, PR)]],
                            b_bufs[q2].at[sl], gsems[q2]))
                    return ds

                pend_g = [fire_blk(0), fire_blk(1)]
                for b in range(SCB):
                    q2, q3 = b % 2, b % 3
                    for d in pend_g[q2]:
                        d.wait()
                    if pend_store[q3] is not None:
                        for d in pend_store[q3]:
                            d.wait()
                        pend_store[q3] = None
                    vadd(c_bufs[q3], a_bufs[q2], b_bufs[q2])
                    rb = row0 + b * PR
                    pend_store[q3] = [pltpu.async_copy(
                        c_bufs[q3].at[pl.ds(j * PR, PR)],
                        pre_hbm.at[pl.ds(rb, PR), pl.ds(j * H, H)],
                        ssems[q3]) for j in range(4)]
                    if b + 2 < SCB:
                        pend_g[q2] = fire_blk(b + 2)
                for q3 in range(3):
                    if pend_store[q3] is not None:
                        for d in pend_store[q3]:
                            d.wait()

            return carry

        lax.fori_loop(0, G_ITERS, sc_body, 0)

    return _gather_add_body


def _make_scatter_add_body(r0):
    def _scatter_add_body(eup_hbm, src_hbm, nup_hbm,
                          ixs, l0, l1, l2, e0, e1, e2, accum,
                          isem, d0sem, d1sem, d2sem, c0sem, c1sem, c2sem):
        cid = lax.axis_index("c")
        sid = lax.axis_index("s")
        base_node = cid * HALF
        e_vs = (e0, e1, e2)
        lidx = (l0, l1, l2)
        lsems = (d0sem, d1sem, d2sem)
        scsems = (c0sem, c1sem, c2sem)

        def z_body(i, carry):
            zero = jnp.zeros((16,), jnp.float32)
            e0[i, pl.ds(0, 16)] = zero
            e0[i, pl.ds(16, 16)] = zero
            return carry

        lax.fori_loop(0, BLKS, z_body, 0, unroll=8)
        rows_per_s = AROWS // NS
        zbase = sid * rows_per_s
        done = 0
        while done < rows_per_s:
            chunk = min(BLKS, rows_per_s - done)
            pltpu.sync_copy(e0.at[pl.ds(0, chunk)],
                            accum.at[pl.ds(zbase + done, chunk)])
            done += chunk
        plsc.subcore_barrier()

        def transform(ib, q):
            def tr_body(k, carry):
                for u in range(2):
                    kk = k * 2 + u
                    j, r = kk // 4, kk % 4
                    v = ixs[j, pl.ds(ib * PRS + r * 16, 16)]
                    li = v - base_node
                    oob = (li < 0) | (li >= HALF)
                    dummy = HALF + (v & (PAD - 1))
                    lidx[q][j, pl.ds(r * 16, 16)] = jnp.where(oob, dummy, li)
                return carry
            lax.fori_loop(0, 8, tr_body, 0)

        def sc_body(t, carry):
            scid = sid + NS * t

            @pl.when(scid < NSCS)
            def _():
                row0 = scid * (SCBS * PRS)
                ids = [pltpu.async_copy(
                    src_hbm.at[pl.ds(j * EP + r0 + row0, SCBS * PRS)],
                    ixs.at[j], isem)
                    for j in range(4)]
                for d in ids:
                    d.wait()

                pend_sc = [None, None, None]

                def fire_load(b):
                    q = b % 3
                    if pend_sc[q] is not None:
                        for d in pend_sc[q]:
                            d.wait()
                        pend_sc[q] = None
                    rb = row0 + b * PRS
                    return [pltpu.async_copy(
                        eup_hbm.at[pl.ds(rb, PRS), pl.ds(j * H, H)],
                        e_vs[q].at[pl.ds(j * PRS, PRS)], lsems[q])
                        for j in range(4)]

                pend_l = [fire_load(0), fire_load(1), None]
                for b in range(SCBS):
                    q = b % 3
                    for d in pend_l[q]:
                        d.wait()
                    transform(b, q)
                    pend_sc[q] = [pltpu.async_copy(
                        e_vs[q].at[pl.ds(j * PRS, PRS)],
                        accum.at[lidx[q].at[j]], scsems[q], add=True)
                        for j in range(4)]
                    if b + 2 < SCBS:
                        pend_l[(b + 2) % 3] = fire_load(b + 2)
                for q in range(3):
                    if pend_sc[q] is not None:
                        for d in pend_sc[q]:
                            d.wait()

            return carry

        lax.fori_loop(0, S_ITERS, sc_body, 0)

        @pl.when(sid < TAILS)
        def _():
            g = NSCS * SCBS + sid
            rb = g * PRS
            for j in range(4):
                pltpu.sync_copy(src_hbm.at[pl.ds(j * EP + r0 + rb, PRS)],
                                ixs.at[j, pl.ds(0, PRS)])
                pltpu.sync_copy(eup_hbm.at[pl.ds(rb, PRS), pl.ds(j * H, H)],
                                e0.at[pl.ds(j * PRS, PRS)])
            transform(0, 0)
            for j in range(4):
                pltpu.sync_copy(e0.at[pl.ds(j * PRS, PRS)],
                                accum.at[l0.at[j]], add=True)

        plsc.subcore_barrier()

        rows_out = HALF // NS
        obase = sid * rows_out
        pltpu.sync_copy(accum.at[pl.ds(obase, rows_out)],
                        nup_hbm.at[pl.ds(base_node + obase, rows_out)])

    return _scatter_add_body


def _tc_call(body, grid, in_specs, out_specs, out_shape):
    return pl.pallas_call(
        body, grid=grid, in_specs=in_specs, out_specs=out_specs,
        out_shape=out_shape)


def kernel(n, e, e_i, batch, We1, be1, We2, be2, Wn1, bn1, Wn2, bn2):
    del batch
    src = e_i[0]
    dst = e_i[1]
    W2bd = block_diag(*([We2] * 4))
    b2t = jnp.tile(be2, 4).reshape(1, 128)

    BN = 2000
    A, B = _tc_call(
        _node_proj_body, (N // BN,),
        [pl.BlockSpec((BN, F), lambda i: (i, 0)),
         pl.BlockSpec((F, H), lambda i: (0, 0)),
         pl.BlockSpec((F, H), lambda i: (0, 0))],
        [pl.BlockSpec((BN, H), lambda i: (i, 0)),
         pl.BlockSpec((BN, H), lambda i: (i, 0))],
        [jax.ShapeDtypeStruct((N, H), jnp.float32),
         jax.ShapeDtypeStruct((N, H), jnp.float32)])(
            n, We1[0:F], We1[F + 10:])

    BE4 = 4000
    NB4 = EP // BE4
    CB4 = CEP // BE4
    be1r = be1.reshape(1, H)
    We1m = We1[F:F + 10]

    gather_scratch = [
        pltpu.VMEM((4, SCB * PR), jnp.int32),
        pltpu.VMEM((4, SCB * PR), jnp.int32),
        pltpu.VMEM((BLK, H), jnp.float32),
        pltpu.VMEM((BLK, H), jnp.float32),
        pltpu.VMEM((BLK, H), jnp.float32),
        pltpu.VMEM((BLK, H), jnp.float32),
        pltpu.VMEM((BLK, H), jnp.float32),
        pltpu.VMEM((BLK, H), jnp.float32),
        pltpu.VMEM((BLK, H), jnp.float32),
        pltpu.SemaphoreType.DMA,
        pltpu.SemaphoreType.DMA,
        pltpu.SemaphoreType.DMA,
        pltpu.SemaphoreType.DMA,
        pltpu.SemaphoreType.DMA,
        pltpu.SemaphoreType.DMA,
    ]
    scatter_scratch = [
        pltpu.VMEM((4, SCBS * PRS), jnp.int32),
        pltpu.VMEM((4, PRS), jnp.int32),
        pltpu.VMEM((4, PRS), jnp.int32),
        pltpu.VMEM((4, PRS), jnp.int32),
        pltpu.VMEM((BLKS, H), jnp.float32),
        pltpu.VMEM((BLKS, H), jnp.float32),
        pltpu.VMEM((BLKS, H), jnp.float32),
        pltpu.VMEM_SHARED((AROWS, H), jnp.float32),
        pltpu.SemaphoreType.DMA,
        pltpu.SemaphoreType.DMA,
        pltpu.SemaphoreType.DMA,
        pltpu.SemaphoreType.DMA,
        pltpu.SemaphoreType.DMA,
        pltpu.SemaphoreType.DMA,
        pltpu.SemaphoreType.DMA,
    ]

    nups = []
    for k in range(NCHUNK):
        C4 = _tc_call(
            _edge_proj_body, (CB4,),
            [pl.BlockSpec((BE4, 10), lambda i, k=k: (k * CB4 + i, 0)),
             pl.BlockSpec((BE4, 10), lambda i, k=k: (NB4 + k * CB4 + i, 0)),
             pl.BlockSpec((BE4, 10),
                          lambda i, k=k: (2 * NB4 + k * CB4 + i, 0)),
             pl.BlockSpec((BE4, 10),
                          lambda i, k=k: (3 * NB4 + k * CB4 + i, 0)),
             pl.BlockSpec((10, H), lambda i: (0, 0)),
             pl.BlockSpec((1, H), lambda i: (0, 0))],
            pl.BlockSpec((BE4, 128), lambda i: (i, 0)),
            jax.ShapeDtypeStruct((CEP, 128), jnp.float32))(
                e, e, e, e, We1m, be1r)

        gather_add = pl.kernel(
            _make_gather_add_body(k * CEP),
            out_type=jax.ShapeDtypeStruct((CEP, 128), jnp.float32),
            mesh=_mesh,
            compiler_params=_sc_params,
            scratch_types=gather_scratch)
        pre4 = gather_add(A, B, src, dst)

        e_up4 = _tc_call(
            _edge_mlp2_body, (CB4,),
            [pl.BlockSpec((BE4, 128), lambda i: (i, 0)),
             pl.BlockSpec((BE4, 128), lambda i: (i, 0)),
             pl.BlockSpec((128, 128), lambda i: (0, 0)),
             pl.BlockSpec((1, 128), lambda i: (0, 0))],
            pl.BlockSpec((BE4, 128), lambda i: (i, 0)),
            jax.ShapeDtypeStruct((CEP, 128), jnp.float32))(
                pre4, C4, W2bd, b2t)

        scatter = pl.kernel(
            _make_scatter_add_body(k * CEP),
            out_type=jax.ShapeDtypeStruct((N, H), jnp.float32),
            mesh=_mesh,
            compiler_params=_sc_params,
            scratch_types=scatter_scratch)
        nups.append(scatter(e_up4, src))

    out = _tc_call(
        _node_mlp_body, (N // BN,),
        [pl.BlockSpec((BN, H), lambda i: (i, 0)),
         pl.BlockSpec((BN, H), lambda i: (i, 0)),
         pl.BlockSpec((BN, H), lambda i: (i, 0)),
         pl.BlockSpec((BN, H), lambda i: (i, 0)),
         pl.BlockSpec((BN, H), lambda i: (i, 0)),
         pl.BlockSpec((BN, F), lambda i: (i, 0)),
         pl.BlockSpec((H, H), lambda i: (0, 0)),
         pl.BlockSpec((F, H), lambda i: (0, 0)),
         pl.BlockSpec((1, H), lambda i: (0, 0)),
         pl.BlockSpec((H, 1), lambda i: (0, 0)),
         pl.BlockSpec((1, 1), lambda i: (0, 0))],
        pl.BlockSpec((BN, 1), lambda i: (i, 0)),
        jax.ShapeDtypeStruct((N, 1), jnp.float32))(
            nups[0], nups[1], nups[2], nups[3], nups[4], n,
            Wn1[0:H], Wn1[H:], bn1.reshape(1, H), Wn2,
            bn2.reshape(1, 1))
    return out


# speedup vs baseline: 1.3419x; 1.0210x over previous
import jax
import jax.numpy as jnp
from jax import lax
from jax.experimental import pallas as pl
from jax.experimental.pallas import tpu as pltpu
from jax.experimental.pallas import tpu_sc as plsc
from jax.scipy.linalg import block_diag

N = 100000
E = 1600000
F = 39
H = 32
EP = E // 4
NCHUNK = 5
CEP = EP // NCHUNK

NC = 2
NS = 16
NW = NC * NS

BLK = 512
PR = BLK // 4
NBLK = CEP // PR
SCB = 5
NSC = NBLK // SCB
G_ITERS = -(-NSC // NW)
HALF = N // NC
PAD = 64
AROWS = HALF + PAD

BLKS = 256
PRS = BLKS // 4
SCBS = 8
NBLKS = CEP // PRS
NSCS = NBLKS // SCBS
TAILS = NBLKS - NSCS * SCBS
S_ITERS = -(-NSCS // NS)

_mesh = plsc.VectorSubcoreMesh(
    core_axis_name="c", subcore_axis_name="s", num_cores=NC, num_subcores=NS)
_sc_params = pltpu.CompilerParams(use_tc_tiling_on_sc=False)


def _node_proj_body(n_ref, wsrc_ref, wdst_ref, a_ref, b_ref):
    x = n_ref[...]
    a_ref[...] = jnp.dot(x, wsrc_ref[...], preferred_element_type=jnp.float32)
    b_ref[...] = jnp.dot(x, wdst_ref[...], preferred_element_type=jnp.float32)


def _edge_proj_body(e0_ref, e1_ref, e2_ref, e3_ref, w_ref, b_ref, c_ref):
    for j, e_ref in enumerate((e0_ref, e1_ref, e2_ref, e3_ref)):
        c_ref[:, j * H:(j + 1) * H] = (
            jnp.dot(e_ref[...], w_ref[...], preferred_element_type=jnp.float32)
            + b_ref[...])


def _edge_mlp2_body(pre_ref, c_ref, w_ref, b_ref, o_ref):
    h = jnp.tanh(pre_ref[...] + c_ref[...])
    o_ref[...] = jnp.tanh(
        jnp.dot(h, w_ref[...], preferred_element_type=jnp.float32) + b_ref[...])


def _node_mlp_body(nu0_ref, nu1_ref, nu2_ref, nu3_ref, nu4_ref, n_ref,
                   w1a_ref, w1b_ref, b1_ref, w2_ref, b2_ref, o_ref):
    nu = (nu0_ref[...] + nu1_ref[...] + nu2_ref[...] + nu3_ref[...]
          + nu4_ref[...])
    z = jnp.tanh(
        jnp.dot(nu, w1a_ref[...], preferred_element_type=jnp.float32)
        + jnp.dot(n_ref[...], w1b_ref[...], preferred_element_type=jnp.float32)
        + b1_ref[...])
    o_ref[...] = (
        jnp.dot(z, w2_ref[...], preferred_element_type=jnp.float32)
        + b2_ref[...])


def _make_gather_add_body(r0):
    def _gather_add_body(a_hbm, b_hbm, ei_hbm, pre_hbm,
                         ixs, ixd, a0, a1, b0, b1, c0, c1, c2,
                         isem, g0sem, g1sem, s0sem, s1sem, s2sem):
        wid = lax.axis_index("s") * NC + lax.axis_index("c")
        a_bufs = (a0, a1)
        b_bufs = (b0, b1)
        c_bufs = (c0, c1, c2)
        gsems = (g0sem, g1sem)
        ssems = (s0sem, s1sem, s2sem)

        def vadd(c_v, a_v, b_v):
            def add_body(i, carry2):
                for h in range(2):
                    sl2 = pl.ds(h * 16, 16)
                    c_v[i, sl2] = a_v[i, sl2] + b_v[i, sl2]
                return carry2
            lax.fori_loop(0, BLK, add_body, 0, unroll=4)

        def sc_body(t, carry):
            scid = wid + NW * t

            @pl.when(scid < NSC)
            def _():
                row0 = scid * (SCB * PR)
                ids = []
                for j in range(4):
                    ids.append(pltpu.async_copy(
                        ei_hbm.at[0, pl.ds(j * EP + r0 + row0, SCB * PR)],
                        ixs.at[j], isem))
                    ids.append(pltpu.async_copy(
                        ei_hbm.at[1, pl.ds(j * EP + r0 + row0, SCB * PR)],
                        ixd.at[j], isem))
                for d in ids:
                    d.wait()

                pend_store = [None, None, None]

                def fire_blk(b):
                    q2 = b % 2
                    rb = row0 + b * PR
                    ds = []
                    for j in range(4):
                        sl = pl.ds(j * PR, PR)
                        ds.append(pltpu.async_copy(
                            a_hbm.at[ixs.at[j, pl.ds(b * PR, PR)]],
                            a_bufs[q2].at[sl], gsems[q2]))
                        ds.append(pltpu.async_copy(
                            b_hbm.at[ixd.at[j, pl.ds(b * PR, PR)]],
                            b_bufs[q2].at[sl], gsems[q2]))
                    return ds

                pend_g = [fire_blk(0), fire_blk(1)]
                for b in range(SCB):
                    q2, q3 = b % 2, b % 3
                    for d in pend_g[q2]:
                        d.wait()
                    if pend_store[q3] is not None:
                        for d in pend_store[q3]:
                            d.wait()
                        pend_store[q3] = None
                    vadd(c_bufs[q3], a_bufs[q2], b_bufs[q2])
                    rb = row0 + b * PR
                    pend_store[q3] = [pltpu.async_copy(
                        c_bufs[q3].at[pl.ds(j * PR, PR)],
                        pre_hbm.at[pl.ds(rb, PR), pl.ds(j * H, H)],
                        ssems[q3]) for j in range(4)]
                    if b + 2 < SCB:
                        pend_g[q2] = fire_blk(b + 2)
                for q3 in range(3):
                    if pend_store[q3] is not None:
                        for d in pend_store[q3]:
                            d.wait()

            return carry

        lax.fori_loop(0, G_ITERS, sc_body, 0)

    return _gather_add_body


def _make_scatter_add_body(r0, final=False):
    def _scatter_add_body(eup_hbm, ei_hbm, *rest):
        if final:
            p0, p1, p2, p3, nup_hbm = rest[:5]
            (ixs, l0, l1, l2, e0, e1, e2, pa, pb, accum,
             isem, d0sem, d1sem, d2sem, c0sem, c1sem, c2sem) = rest[5:]
        else:
            nup_hbm = rest[0]
            (ixs, l0, l1, l2, e0, e1, e2, accum,
             isem, d0sem, d1sem, d2sem, c0sem, c1sem, c2sem) = rest[1:]
        cid = lax.axis_index("c")
        sid = lax.axis_index("s")
        base_node = cid * HALF
        e_vs = (e0, e1, e2)
        lidx = (l0, l1, l2)
        lsems = (d0sem, d1sem, d2sem)
        scsems = (c0sem, c1sem, c2sem)

        def z_body(i, carry):
            zero = jnp.zeros((16,), jnp.float32)
            e0[i, pl.ds(0, 16)] = zero
            e0[i, pl.ds(16, 16)] = zero
            return carry

        lax.fori_loop(0, BLKS, z_body, 0, unroll=8)
        rows_per_s = AROWS // NS
        zbase = sid * rows_per_s
        done = 0
        while done < rows_per_s:
            chunk = min(BLKS, rows_per_s - done)
            pltpu.sync_copy(e0.at[pl.ds(0, chunk)],
                            accum.at[pl.ds(zbase + done, chunk)])
            done += chunk
        plsc.subcore_barrier()

        def transform(ib, q):
            def tr_body(k, carry):
                for u in range(2):
                    kk = k * 2 + u
                    j, r = kk // 4, kk % 4
                    v = ixs[j, pl.ds(ib * PRS + r * 16, 16)]
                    li = v - base_node
                    oob = (li < 0) | (li >= HALF)
                    dummy = HALF + (v & (PAD - 1))
                    lidx[q][j, pl.ds(r * 16, 16)] = jnp.where(oob, dummy, li)
                return carry
            lax.fori_loop(0, 8, tr_body, 0)

        def sc_body(t, carry):
            scid = sid + NS * t

            @pl.when(scid < NSCS)
            def _():
                row0 = scid * (SCBS * PRS)
                ids = [pltpu.async_copy(
                    ei_hbm.at[0, pl.ds(j * EP + r0 + row0, SCBS * PRS)],
                    ixs.at[j], isem)
                    for j in range(4)]
                for d in ids:
                    d.wait()

                pend_sc = [None, None, None]

                def fire_load(b):
                    q = b % 3
                    if pend_sc[q] is not None:
                        for d in pend_sc[q]:
                            d.wait()
                        pend_sc[q] = None
                    rb = row0 + b * PRS
                    return [pltpu.async_copy(
                        eup_hbm.at[pl.ds(rb, PRS), pl.ds(j * H, H)],
                        e_vs[q].at[pl.ds(j * PRS, PRS)], lsems[q])
                        for j in range(4)]

                pend_l = [fire_load(0), fire_load(1), None]
                for b in range(SCBS):
                    q = b % 3
                    for d in pend_l[q]:
                        d.wait()
                    transform(b, q)
                    pend_sc[q] = [pltpu.async_copy(
                        e_vs[q].at[pl.ds(j * PRS, PRS)],
                        accum.at[lidx[q].at[j]], scsems[q], add=True)
                        for j in range(4)]
                    if b + 2 < SCBS:
                        pend_l[(b + 2) % 3] = fire_load(b + 2)
                for q in range(3):
                    if pend_sc[q] is not None:
                        for d in pend_sc[q]:
                            d.wait()

            return carry

        lax.fori_loop(0, S_ITERS, sc_body, 0)

        @pl.when(sid < TAILS)
        def _():
            g = NSCS * SCBS + sid
            rb = g * PRS
            for j in range(4):
                pltpu.sync_copy(ei_hbm.at[0, pl.ds(j * EP + r0 + rb, PRS)],
                                ixs.at[j, pl.ds(0, PRS)])
                pltpu.sync_copy(eup_hbm.at[pl.ds(rb, PRS), pl.ds(j * H, H)],
                                e0.at[pl.ds(j * PRS, PRS)])
            transform(0, 0)
            for j in range(4):
                pltpu.sync_copy(e0.at[pl.ds(j * PRS, PRS)],
                                accum.at[l0.at[j]], add=True)

        plsc.subcore_barrier()

        rows_out = HALF // NS
        obase = sid * rows_out
        if not final:
            pltpu.sync_copy(accum.at[pl.ds(obase, rows_out)],
                            nup_hbm.at[pl.ds(base_node + obase, rows_out)])
        else:
            done = 0
            while done < rows_out:
                chunk = min(BLKS, rows_out - done)
                lo = obase + done
                go = base_node + obase + done
                ds = [pltpu.async_copy(accum.at[pl.ds(lo, chunk)],
                                       e0.at[pl.ds(0, chunk)], d0sem)]
                for p, buf, sem in ((p0, e1, d1sem), (p1, e2, d2sem),
                                    (p2, pa, c0sem), (p3, pb, c1sem)):
                    ds.append(pltpu.async_copy(p.at[pl.ds(go, chunk)],
                                               buf.at[pl.ds(0, chunk)], sem))
                for d in ds:
                    d.wait()

                def add_body(i, carry):
                    for h in range(2):
                        sl2 = pl.ds(h * 16, 16)
                        e0[i, sl2] = (e0[i, sl2] + e1[i, sl2] + e2[i, sl2]
                                      + pa[i, sl2] + pb[i, sl2])
                    return carry
                lax.fori_loop(0, chunk, add_body, 0, unroll=4)
                pltpu.sync_copy(e0.at[pl.ds(0, chunk)],
                                nup_hbm.at[pl.ds(go, chunk)])
                done += chunk

    return _scatter_add_body


def _tc_call(body, grid, in_specs, out_specs, out_shape):
    return pl.pallas_call(
        body, grid=grid, in_specs=in_specs, out_specs=out_specs,
        out_shape=out_shape)


def kernel(n, e, e_i, batch, We1, be1, We2, be2, Wn1, bn1, Wn2, bn2):
    del batch
    W2bd = block_diag(*([We2] * 4))
    b2t = jnp.tile(be2, 4).reshape(1, 128)

    BN = 2000
    A, B = _tc_call(
        _node_proj_body, (N // BN,),
        [pl.BlockSpec((BN, F), lambda i: (i, 0)),
         pl.BlockSpec((F, H), lambda i: (0, 0)),
         pl.BlockSpec((F, H), lambda i: (0, 0))],
        [pl.BlockSpec((BN, H), lambda i: (i, 0)),
         pl.BlockSpec((BN, H), lambda i: (i, 0))],
        [jax.ShapeDtypeStruct((N, H), jnp.float32),
         jax.ShapeDtypeStruct((N, H), jnp.float32)])(
            n, We1[0:F], We1[F + 10:])

    BE4 = 4000
    NB4 = EP // BE4
    CB4 = CEP // BE4
    be1r = be1.reshape(1, H)
    We1m = We1[F:F + 10]

    gather_scratch = [
        pltpu.VMEM((4, SCB * PR), jnp.int32),
        pltpu.VMEM((4, SCB * PR), jnp.int32),
        pltpu.VMEM((BLK, H), jnp.float32),
        pltpu.VMEM((BLK, H), jnp.float32),
        pltpu.VMEM((BLK, H), jnp.float32),
        pltpu.VMEM((BLK, H), jnp.float32),
        pltpu.VMEM((BLK, H), jnp.float32),
        pltpu.VMEM((BLK, H), jnp.float32),
        pltpu.VMEM((BLK, H), jnp.float32),
        pltpu.SemaphoreType.DMA,
        pltpu.SemaphoreType.DMA,
        pltpu.SemaphoreType.DMA,
        pltpu.SemaphoreType.DMA,
        pltpu.SemaphoreType.DMA,
        pltpu.SemaphoreType.DMA,
    ]
    scatter_scratch = [
        pltpu.VMEM((4, SCBS * PRS), jnp.int32),
        pltpu.VMEM((4, PRS), jnp.int32),
        pltpu.VMEM((4, PRS), jnp.int32),
        pltpu.VMEM((4, PRS), jnp.int32),
        pltpu.VMEM((BLKS, H), jnp.float32),
        pltpu.VMEM((BLKS, H), jnp.float32),
        pltpu.VMEM((BLKS, H), jnp.float32),
        pltpu.VMEM_SHARED((AROWS, H), jnp.float32),
        pltpu.SemaphoreType.DMA,
        pltpu.SemaphoreType.DMA,
        pltpu.SemaphoreType.DMA,
        pltpu.SemaphoreType.DMA,
        pltpu.SemaphoreType.DMA,
        pltpu.SemaphoreType.DMA,
        pltpu.SemaphoreType.DMA,
    ]

    nups = []
    for k in range(NCHUNK):
        C4 = _tc_call(
            _edge_proj_body, (CB4,),
            [pl.BlockSpec((BE4, 10), lambda i, k=k: (k * CB4 + i, 0)),
             pl.BlockSpec((BE4, 10), lambda i, k=k: (NB4 + k * CB4 + i, 0)),
             pl.BlockSpec((BE4, 10),
                          lambda i, k=k: (2 * NB4 + k * CB4 + i, 0)),
             pl.BlockSpec((BE4, 10),
                          lambda i, k=k: (3 * NB4 + k * CB4 + i, 0)),
             pl.BlockSpec((10, H), lambda i: (0, 0)),
             pl.BlockSpec((1, H), lambda i: (0, 0))],
            pl.BlockSpec((BE4, 128), lambda i: (i, 0)),
            jax.ShapeDtypeStruct((CEP, 128), jnp.float32))(
                e, e, e, e, We1m, be1r)

        gather_add = pl.kernel(
            _make_gather_add_body(k * CEP),
            out_type=jax.ShapeDtypeStruct((CEP, 128), jnp.float32),
            mesh=_mesh,
            compiler_params=_sc_params,
            scratch_types=gather_scratch)
        pre4 = gather_add(A, B, e_i)

        e_up4 = _tc_call(
            _edge_mlp2_body, (CB4,),
            [pl.BlockSpec((BE4, 128), lambda i: (i, 0)),
             pl.BlockSpec((BE4, 128), lambda i: (i, 0)),
             pl.BlockSpec((128, 128), lambda i: (0, 0)),
             pl.BlockSpec((1, 128), lambda i: (0, 0))],
            pl.BlockSpec((BE4, 128), lambda i: (i, 0)),
            jax.ShapeDtypeStruct((CEP, 128), jnp.float32))(
                pre4, C4, W2bd, b2t)

        scatter = pl.kernel(
            _make_scatter_add_body(k * CEP),
            out_type=jax.ShapeDtypeStruct((N, H), jnp.float32),
            mesh=_mesh,
            compiler_params=_sc_params,
            scratch_types=scatter_scratch)
        nups.append(scatter(e_up4, e_i))

    out = _tc_call(
        _node_mlp_body, (N // BN,),
        [pl.BlockSpec((BN, H), lambda i: (i, 0)),
         pl.BlockSpec((BN, H), lambda i: (i, 0)),
         pl.BlockSpec((BN, H), lambda i: (i, 0)),
         pl.BlockSpec((BN, H), lambda i: (i, 0)),
         pl.BlockSpec((BN, H), lambda i: (i, 0)),
         pl.BlockSpec((BN, F), lambda i: (i, 0)),
         pl.BlockSpec((H, H), lambda i: (0, 0)),
         pl.BlockSpec((F, H), lambda i: (0, 0)),
         pl.BlockSpec((1, H), lambda i: (0, 0)),
         pl.BlockSpec((H, 1), lambda i: (0, 0)),
         pl.BlockSpec((1, 1), lambda i: (0, 0))],
        pl.BlockSpec((BN, 1), lambda i: (i, 0)),
        jax.ShapeDtypeStruct((N, 1), jnp.float32))(
            nups[0], nups[1], nups[2], nups[3], nups[4], n,
            Wn1[0:H], Wn1[H:], bn1.reshape(1, H), Wn2,
            bn2.reshape(1, 1))
    return out
